# baseline (speedup 1.0000x reference)
import jax
import jax.numpy as jnp
from jax import lax
from jax.experimental import pallas as pl
from jax.experimental.pallas import tpu as pltpu
from jax.experimental.pallas import tpu_sc as plsc

D = 64
C = 128
PW = 2 * D
V = 1000000
QFULL = V // C
SROWS = QFULL * D + D
NC = 2
NS = 16
NW = NC * NS
HIST = 50
BATCH = 16384
ITEMS = HIST * (BATCH // C)
IPW = ITEMS // NW
TP = C + 1
NBUF = 2
NG = IPW // NBUF
JA = QFULL // NW


def _relayout_body(wt_hbm, tailt_hbm, scr_hbm, g_v, t_v, gsems, osems):
    wid = lax.axis_index("s") * NC + lax.axis_index("c")
    iota16 = lax.iota(jnp.int32, 16)

    def gin(j, b):
        q = j * NW + wid
        return pltpu.make_async_copy(
            wt_hbm.at[pl.ds(0, D), pl.ds(q * C, C)], g_v.at[b], gsems.at[b])

    def gout(j, b):
        q = j * NW + wid
        return pltpu.make_async_copy(
            t_v.at[b, pl.ds(0, D), pl.ds(0, PW)],
            scr_hbm.at[pl.ds(q * D, D)], osems.at[b])

    def transpose(b):
        @plsc.parallel_loop(0, D, step=1, unroll=4)
        def _(d):
            dvec = jnp.zeros((16,), jnp.int32) + d
            for c in range(8):
                lvec = iota16 + 16 * c
                vec = g_v[b, d, pl.ds(c * 16, 16)]
                plsc.store_scatter(
                    t_v.at[b], [lvec % D, (lvec // D) * D + dvec], vec)

    for b in range(NBUF):
        gin(b, b).start()

    def group(g, carry):
        for b in range(NBUF):
            j = g * NBUF + b
            gin(j, b).wait()

            @pl.when(g > 0)
            def _():
                gout(j - NBUF, b).wait()

            transpose(b)
            nj = j + NBUF

            @pl.when((nj < JA) | ((nj == JA) & (wid < 4)))
            def _():
                gin(nj, b).start()

            gout(j, b).start()
        return carry

    lax.fori_loop(0, JA // NBUF, group, 0)

    @pl.when(wid < 4)
    def _():
        gin(JA, 0).wait()
        gout(JA - NBUF, 0).wait()
        transpose(0)
        gout(JA, 0).start()
        gout(JA, 0).wait()

    @pl.when(wid >= 4)
    def _():
        gout(JA - NBUF, 0).wait()

    gout(JA - 1, 1).wait()

    @pl.when(wid == 0)
    def _():
        pltpu.sync_copy(tailt_hbm, g_v.at[0])

        @plsc.parallel_loop(0, D, step=1, unroll=4)
        def _(d):
            dvec = jnp.zeros((16,), jnp.int32) + d
            for c in range(4):
                lvec = iota16 + 16 * c
                vec = g_v[0, d, pl.ds(c * 16, 16)]
                plsc.store_scatter(t_v.at[0], [lvec, dvec], vec)

        pltpu.sync_copy(t_v.at[0, pl.ds(0, D), pl.ds(0, PW)],
                        scr_hbm.at[pl.ds(QFULL * D, D)])


def _emb_body(idxp_hbm, offs_hbm, tab_hbm, out_hbm,
              idx_v, off_v, g_v, t_v, gsems, osems):
    wid = lax.axis_index("s") * NC + lax.axis_index("c")
    base = wid * IPW
    pltpu.sync_copy(idxp_hbm.at[wid], idx_v)
    pltpu.sync_copy(offs_hbm.at[wid], off_v)

    def gather(jl, b):
        return pltpu.make_async_copy(
            tab_hbm.at[idx_v.at[jl]], g_v.at[b], gsems.at[b])

    def outcopy(jl, b):
        t = base + jl
        h = t // C
        tb = t % C
        return pltpu.make_async_copy(
            t_v.at[b, pl.ds(0, 8), pl.ds(0, 8), pl.ds(0, C)],
            out_hbm.at[h, pl.ds(0, 8), tb], osems.at[b])

    iota16 = lax.iota(jnp.int32, 16)

    def transpose(jl, b):
        t3 = t_v.at[b]

        @plsc.parallel_loop(0, C // 16, step=1, unroll=2)
        def _(s):
            offv = off_v[jl, pl.ds(s * 16, 16)]
            for k in range(16):
                sb = s * 16 + k
                off = offv[k]
                sbvec = jnp.zeros((16,), jnp.int32) + sb
                for c in range(4):
                    vec = g_v[b, sb, pl.ds(off + c * 16, 16)]
                    plsc.store_scatter(
                        t3, [(iota16 + 16 * c) // 8, (iota16 + 16 * c) % 8,
                             sbvec], vec)

    for b in range(NBUF):
        gather(b, b).start()

    def group(g, carry):
        for b in range(NBUF):
            jl = g * NBUF + b
            gather(jl, b).wait()

            @pl.when(g > 0)
            def _():
                outcopy(jl - NBUF, b).wait()

            transpose(jl, b)

            @pl.when(jl + NBUF < IPW)
            def _():
                gather(jl + NBUF, b).start()

            outcopy(jl, b).start()
        return carry

    lax.fori_loop(0, NG, group, 0)

    last = (NG - 1) * NBUF
    for b in range(NBUF):
        outcopy(last + b, b).wait()


def kernel(input_, weight):
    bsz, hist = input_.shape
    nb = bsz // C
    mesh = plsc.VectorSubcoreMesh(core_axis_name="c", subcore_axis_name="s")
    params = pltpu.CompilerParams(
        use_tc_tiling_on_sc=True, needs_layout_passes=False)

    ka = pl.kernel(
        _relayout_body,
        mesh=mesh,
        out_type=jax.ShapeDtypeStruct((SROWS, PW), jnp.float32),
        scratch_types=[
            pltpu.VMEM((NBUF, D, C), jnp.float32),
            pltpu.VMEM((NBUF, D, TP), jnp.float32),
            pltpu.SemaphoreType.DMA((NBUF,)),
            pltpu.SemaphoreType.DMA((NBUF,)),
        ],
        compiler_params=params,
    )
    kb = pl.kernel(
        _emb_body,
        mesh=mesh,
        out_type=jax.ShapeDtypeStruct((hist, 8, nb, 8, C), jnp.float32),
        scratch_types=[
            pltpu.VMEM((IPW, C), jnp.int32),
            pltpu.VMEM((IPW, C), jnp.int32),
            pltpu.VMEM((NBUF, C, PW), jnp.float32),
            pltpu.VMEM((NBUF, 8, 8, TP), jnp.float32),
            pltpu.SemaphoreType.DMA((NBUF,)),
            pltpu.SemaphoreType.DMA((NBUF,)),
        ],
        compiler_params=params,
    )

    it = input_.T.astype(jnp.int32)
    idxp = ((it >> 7) * D + (it & (D - 1))).reshape(NW, IPW, C)
    offs = (it & D).reshape(NW, IPW, C)
    tailt = jnp.pad(weight[QFULL * C:].T, ((0, 0), (0, D)))
    scratch = ka(weight.T, tailt)
    out = kb(idxp, offs, scratch)
    return out.transpose(2, 4, 0, 1, 3).reshape(bsz, hist, D)


# speedup vs baseline: 2.1690x; 2.1690x over previous
import jax
import jax.numpy as jnp
from jax import lax
from jax.experimental import pallas as pl
from jax.experimental.pallas import tpu as pltpu
from jax.experimental.pallas import tpu_sc as plsc

D = 64
C = 128
NC = 2
NS = 16
NW = NC * NS
NBUF = 4
HIST = 50
BATCH = 16384
ITEMS = HIST * (BATCH // C)
IPW = ITEMS // NW
NG = IPW // NBUF
TP = C + 1


def _emb_body(idxt_hbm, tab_hbm, out_hbm, idx_v, g_v, t_v, gsems, osems):
    wid = lax.axis_index("s") * NC + lax.axis_index("c")
    base = wid * IPW
    pltpu.sync_copy(idxt_hbm.at[wid], idx_v)

    def gather(jl, b):
        return pltpu.make_async_copy(
            tab_hbm.at[idx_v.at[jl]], g_v.at[b], gsems.at[b])

    def outcopy(jl, b):
        t = base + jl
        h = t // C
        tb = t % C
        return pltpu.make_async_copy(
            t_v.at[b, pl.ds(0, 8), pl.ds(0, 8), pl.ds(0, C)],
            out_hbm.at[h, pl.ds(0, 8), tb], osems.at[b])

    iota16 = lax.iota(jnp.int32, 16)

    def transpose(b):
        t3 = t_v.at[b]

        @plsc.parallel_loop(0, C, step=1, unroll=4)
        def _(sb):
            sbvec = jnp.zeros((16,), jnp.int32) + sb
            for c in range(4):
                vec = g_v[b, sb, pl.ds(c * 16, 16)]
                plsc.store_scatter(
                    t3, [(iota16 + 16 * c) // 8, (iota16 + 16 * c) % 8,
                         sbvec], vec)

    for b in range(NBUF):
        gather(b, b).start()

    def group(g, carry):
        for b in range(NBUF):
            jl = g * NBUF + b
            gather(jl, b).wait()

            @pl.when(g > 0)
            def _():
                outcopy(jl - NBUF, b).wait()

            transpose(b)

            @pl.when(jl + NBUF < IPW)
            def _():
                gather(jl + NBUF, b).start()

            outcopy(jl, b).start()
        return carry

    lax.fori_loop(0, NG, group, 0)

    last = (NG - 1) * NBUF
    for b in range(NBUF):
        outcopy(last + b, b).wait()


def kernel(input_, weight):
    bsz, hist = input_.shape
    nb = bsz // C
    idxt = input_.T.astype(jnp.int32).reshape(NW, IPW, C)
    mesh = plsc.VectorSubcoreMesh(core_axis_name="c", subcore_axis_name="s")
    k = pl.kernel(
        _emb_body,
        mesh=mesh,
        out_type=jax.ShapeDtypeStruct((hist, 8, nb, 8, C), jnp.float32),
        scratch_types=[
            pltpu.VMEM((IPW, C), jnp.int32),
            pltpu.VMEM((NBUF, C, D), jnp.float32),
            pltpu.VMEM((NBUF, 8, 8, TP), jnp.float32),
            pltpu.SemaphoreType.DMA((NBUF,)),
            pltpu.SemaphoreType.DMA((NBUF,)),
        ],
        compiler_params=pltpu.CompilerParams(
            use_tc_tiling_on_sc=False, needs_layout_passes=False),
    )
    out = k(idxt, weight)
    return out.transpose(2, 4, 0, 1, 3).reshape(bsz, hist, D)
